# baseline (speedup 1.0000x reference)
import functools

import jax
import jax.numpy as jnp
from jax import lax
from jax.experimental import pallas as pl
from jax.experimental.pallas import tpu as pltpu
from jax.experimental.pallas import tpu_sc as plsc

_VOCAB = 100000
_DIM = 64
_BATCH = 1024
_VB = 512
_NV = (_VOCAB + _VB - 1) // _VB
_NEG = -1e30


def _sc_gather(table, idx):
    info = plsc.get_sparse_core_info()
    nw = info.num_cores * info.num_subcores
    b_per_w = _BATCH // nw
    mesh = plsc.VectorSubcoreMesh(core_axis_name="c", subcore_axis_name="s")

    @functools.partial(
        pl.kernel,
        mesh=mesh,
        out_type=jax.ShapeDtypeStruct((_BATCH, _DIM), jnp.float32),
        scratch_types=[
            pltpu.VMEM((b_per_w,), jnp.int32),
            pltpu.VMEM((b_per_w, _DIM), jnp.float32),
            pltpu.SemaphoreType.DMA,
        ],
        compiler_params=pltpu.CompilerParams(use_tc_tiling_on_sc=False),
    )
    def gather_k(table_hbm, idx_hbm, out_hbm, idx_v, rows_v, sem):
        wid = lax.axis_index("s") * info.num_cores + lax.axis_index("c")
        base = wid * b_per_w
        pltpu.sync_copy(idx_hbm.at[pl.ds(base, b_per_w)], idx_v)
        pltpu.async_copy(table_hbm.at[idx_v], rows_v, sem).wait()
        pltpu.sync_copy(rows_v, out_hbm.at[pl.ds(base, b_per_w)])

    return gather_k(table, idx)


def _logits_block(e_ref, w_ref, b_ref):
    logits = lax.dot_general(
        e_ref[...], w_ref[...], (((1,), (1,)), ((), ())),
        preferred_element_type=jnp.float32,
    )
    return logits + b_ref[...]


def _stats_body(e_ref, w_ref, b_ref, lse_ref, m_s, s_s):
    v = pl.program_id(0)
    logits = _logits_block(e_ref, w_ref, b_ref)
    col = v * _VB + lax.broadcasted_iota(jnp.int32, (1, _VB), 1)
    logits = jnp.where(col < _VOCAB, logits, _NEG)

    @pl.when(v == 0)
    def _():
        m_s[...] = jnp.full_like(m_s, _NEG)
        s_s[...] = jnp.zeros_like(s_s)

    m_old = m_s[...]
    m_new = jnp.maximum(m_old, jnp.max(logits, axis=1, keepdims=True))
    s_new = s_s[...] * jnp.exp(m_old - m_new) + jnp.sum(
        jnp.exp(logits - m_new), axis=1, keepdims=True)
    m_s[...] = m_new
    s_s[...] = s_new

    @pl.when(v == _NV - 1)
    def _():
        lse_ref[...] = m_new + jnp.log(s_new)


def _out_body(e_ref, w_ref, b_ref, lse_ref, o_ref):
    o_ref[...] = _logits_block(e_ref, w_ref, b_ref) - lse_ref[...]


def kernel(target_word, emb_table, W, b):
    embeds = _sc_gather(emb_table, target_word.astype(jnp.int32))
    b2 = b.reshape(1, _VOCAB)

    lse = pl.pallas_call(
        _stats_body,
        grid=(_NV,),
        in_specs=[
            pl.BlockSpec((_BATCH, _DIM), lambda v: (0, 0)),
            pl.BlockSpec((_VB, _DIM), lambda v: (v, 0)),
            pl.BlockSpec((1, _VB), lambda v: (0, v)),
        ],
        out_specs=pl.BlockSpec((_BATCH, 1), lambda v: (0, 0)),
        out_shape=jax.ShapeDtypeStruct((_BATCH, 1), jnp.float32),
        scratch_shapes=[
            pltpu.VMEM((_BATCH, 1), jnp.float32),
            pltpu.VMEM((_BATCH, 1), jnp.float32),
        ],
        compiler_params=pltpu.CompilerParams(
            dimension_semantics=("arbitrary",)),
    )(embeds, W, b2)

    out = pl.pallas_call(
        _out_body,
        grid=(_NV,),
        in_specs=[
            pl.BlockSpec((_BATCH, _DIM), lambda v: (0, 0)),
            pl.BlockSpec((_VB, _DIM), lambda v: (v, 0)),
            pl.BlockSpec((1, _VB), lambda v: (0, v)),
            pl.BlockSpec((_BATCH, 1), lambda v: (0, 0)),
        ],
        out_specs=pl.BlockSpec((_BATCH, _VB), lambda v: (0, v)),
        out_shape=jax.ShapeDtypeStruct((_BATCH, _VOCAB), jnp.float32),
        compiler_params=pltpu.CompilerParams(
            dimension_semantics=("arbitrary",)),
    )(embeds, W, b2, lse)
    return out


# speedup vs baseline: 1.1801x; 1.1801x over previous
import functools

import jax
import jax.numpy as jnp
from jax import lax
from jax.experimental import pallas as pl
from jax.experimental.pallas import tpu as pltpu
from jax.experimental.pallas import tpu_sc as plsc

_VOCAB = 100000
_DIM = 64
_BATCH = 1024
_VB = 2048
_NV = (_VOCAB + _VB - 1) // _VB
_NEG = -1e30


def _sc_gather(table, idx):
    info = plsc.get_sparse_core_info()
    nw = info.num_cores * info.num_subcores
    b_per_w = _BATCH // nw
    mesh = plsc.VectorSubcoreMesh(core_axis_name="c", subcore_axis_name="s")

    @functools.partial(
        pl.kernel,
        mesh=mesh,
        out_type=jax.ShapeDtypeStruct((_BATCH, _DIM), jnp.float32),
        scratch_types=[
            pltpu.VMEM((b_per_w,), jnp.int32),
            pltpu.VMEM((b_per_w, _DIM), jnp.float32),
            pltpu.SemaphoreType.DMA,
        ],
        compiler_params=pltpu.CompilerParams(use_tc_tiling_on_sc=False),
    )
    def gather_k(table_hbm, idx_hbm, out_hbm, idx_v, rows_v, sem):
        wid = lax.axis_index("s") * info.num_cores + lax.axis_index("c")
        base = wid * b_per_w
        pltpu.sync_copy(idx_hbm.at[pl.ds(base, b_per_w)], idx_v)
        pltpu.async_copy(table_hbm.at[idx_v], rows_v, sem).wait()
        pltpu.sync_copy(rows_v, out_hbm.at[pl.ds(base, b_per_w)])

    return gather_k(table, idx)


def _logits_block(e_ref, w_ref, b_ref):
    logits = lax.dot_general(
        e_ref[...].astype(jnp.bfloat16), w_ref[...].astype(jnp.bfloat16),
        (((1,), (1,)), ((), ())),
        preferred_element_type=jnp.float32,
    )
    return logits + b_ref[...]


def _fused_body(e_ref, w_ref, b_ref, o_ref, m_s, s_s, lse_s):
    p = pl.program_id(0)
    v = pl.program_id(1)
    logits = _logits_block(e_ref, w_ref, b_ref)

    @pl.when(p == 0)
    def _stats_phase():
        @pl.when(v == 0)
        def _():
            m_s[...] = jnp.full_like(m_s, _NEG)
            s_s[...] = jnp.zeros_like(s_s)

        col = v * _VB + lax.broadcasted_iota(jnp.int32, (1, _VB), 1)
        lm = jnp.where(col < _VOCAB, logits, _NEG)
        m_old = m_s[...]
        m_new = jnp.maximum(m_old, jnp.max(lm, axis=1, keepdims=True))
        s_new = s_s[...] * jnp.exp(m_old - m_new) + jnp.sum(
            jnp.exp(lm - m_new), axis=1, keepdims=True)
        m_s[...] = m_new
        s_s[...] = s_new

        @pl.when(v == _NV - 1)
        def _():
            lse_s[...] = m_new + jnp.log(s_new)

    @pl.when(p == 1)
    def _write_phase():
        o_ref[...] = logits - lse_s[...]


def kernel(target_word, emb_table, W, b):
    embeds = _sc_gather(emb_table, target_word.astype(jnp.int32))
    b2 = b.reshape(1, _VOCAB)

    out = pl.pallas_call(
        _fused_body,
        grid=(2, _NV),
        in_specs=[
            pl.BlockSpec((_BATCH, _DIM), lambda p, v: (0, 0)),
            pl.BlockSpec((_VB, _DIM), lambda p, v: (v, 0)),
            pl.BlockSpec((1, _VB), lambda p, v: (0, v)),
        ],
        out_specs=pl.BlockSpec((_BATCH, _VB), lambda p, v: (0, p * v)),
        out_shape=jax.ShapeDtypeStruct((_BATCH, _VOCAB), jnp.float32),
        scratch_shapes=[
            pltpu.VMEM((_BATCH, 1), jnp.float32),
            pltpu.VMEM((_BATCH, 1), jnp.float32),
            pltpu.VMEM((_BATCH, 1), jnp.float32),
        ],
        compiler_params=pltpu.CompilerParams(
            dimension_semantics=("arbitrary", "arbitrary")),
    )(embeds, W, b2)
    return out


# speedup vs baseline: 2.0237x; 1.7148x over previous
import functools

import jax
import jax.numpy as jnp
from jax import lax
from jax.experimental import pallas as pl
from jax.experimental.pallas import tpu as pltpu
from jax.experimental.pallas import tpu_sc as plsc

_VOCAB = 100000
_DIM = 64
_BATCH = 1024
_VB = 2048
_NV = (_VOCAB + _VB - 1) // _VB
_NEG = -1e30


def _sc_gather(table, idx):
    info = plsc.get_sparse_core_info()
    nw = info.num_cores * info.num_subcores
    b_per_w = _BATCH // nw
    mesh = plsc.VectorSubcoreMesh(core_axis_name="c", subcore_axis_name="s")

    @functools.partial(
        pl.kernel,
        mesh=mesh,
        out_type=jax.ShapeDtypeStruct((_BATCH, _DIM), jnp.float32),
        scratch_types=[
            pltpu.VMEM((b_per_w,), jnp.int32),
            pltpu.VMEM((b_per_w, _DIM), jnp.float32),
            pltpu.SemaphoreType.DMA,
        ],
        compiler_params=pltpu.CompilerParams(use_tc_tiling_on_sc=False),
    )
    def gather_k(table_hbm, idx_hbm, out_hbm, idx_v, rows_v, sem):
        wid = lax.axis_index("s") * info.num_cores + lax.axis_index("c")
        base = wid * b_per_w
        pltpu.sync_copy(idx_hbm.at[pl.ds(base, b_per_w)], idx_v)
        pltpu.async_copy(table_hbm.at[idx_v], rows_v, sem).wait()
        pltpu.sync_copy(rows_v, out_hbm.at[pl.ds(base, b_per_w)])

    return gather_k(table, idx)


def _logits_t_block(e_ref, wt_ref, b_ref):
    logits_t = lax.dot_general(
        wt_ref[...].astype(jnp.bfloat16), e_ref[...].astype(jnp.bfloat16),
        (((0,), (1,)), ((), ())),
        preferred_element_type=jnp.float32,
    )
    return logits_t + b_ref[...]


def _fused_body(e_ref, wt_ref, b_ref, o_ref, m_s, s_s, lse_s):
    p = pl.program_id(0)
    v = pl.program_id(1)
    logits_t = _logits_t_block(e_ref, wt_ref, b_ref)

    @pl.when(p == 0)
    def _stats_phase():
        @pl.when(v == 0)
        def _():
            m_s[...] = jnp.full_like(m_s, _NEG)
            s_s[...] = jnp.zeros_like(s_s)

        row = v * _VB + lax.broadcasted_iota(jnp.int32, (_VB, 1), 0)
        lm = jnp.where(row < _VOCAB, logits_t, _NEG)
        m_old = m_s[...]
        m_new = jnp.maximum(m_old, jnp.max(lm, axis=0, keepdims=True))
        s_new = s_s[...] * jnp.exp(m_old - m_new) + jnp.sum(
            jnp.exp(lm - m_new), axis=0, keepdims=True)
        m_s[...] = m_new
        s_s[...] = s_new

        @pl.when(v == _NV - 1)
        def _():
            lse_s[...] = m_new + jnp.log(s_new)

    @pl.when(p == 1)
    def _write_phase():
        o_ref[...] = logits_t - lse_s[...]


def kernel(target_word, emb_table, W, b):
    embeds = _sc_gather(emb_table, target_word.astype(jnp.int32))
    Wt = W.T
    b2 = b.reshape(_VOCAB, 1)

    out_t = pl.pallas_call(
        _fused_body,
        grid=(2, _NV),
        in_specs=[
            pl.BlockSpec((_BATCH, _DIM), lambda p, v: (0, 0)),
            pl.BlockSpec((_DIM, _VB), lambda p, v: (0, v)),
            pl.BlockSpec((_VB, 1), lambda p, v: (v, 0)),
        ],
        out_specs=pl.BlockSpec((_VB, _BATCH), lambda p, v: (p * v, 0)),
        out_shape=jax.ShapeDtypeStruct((_VOCAB, _BATCH), jnp.float32),
        scratch_shapes=[
            pltpu.VMEM((1, _BATCH), jnp.float32),
            pltpu.VMEM((1, _BATCH), jnp.float32),
            pltpu.VMEM((1, _BATCH), jnp.float32),
        ],
        compiler_params=pltpu.CompilerParams(
            dimension_semantics=("arbitrary", "arbitrary")),
    )(embeds, Wt, b2)
    return out_t.T


# speedup vs baseline: 2.3788x; 1.1755x over previous
import functools

import jax
import jax.numpy as jnp
from jax import lax
from jax.experimental import pallas as pl
from jax.experimental.pallas import tpu as pltpu
from jax.experimental.pallas import tpu_sc as plsc

_VOCAB = 100000
_DIM = 64
_BATCH = 1024
_VB = 2048
_NV = (_VOCAB + _VB - 1) // _VB
_NEG = -1e30


_NROW = _DIM * _BATCH // 128
_RPW = _NROW // 32


def _sc_gather_t(table_flat, offs):
    info = plsc.get_sparse_core_info()
    mesh = plsc.VectorSubcoreMesh(core_axis_name="c", subcore_axis_name="s")

    @functools.partial(
        pl.kernel,
        mesh=mesh,
        out_type=jax.ShapeDtypeStruct((_NROW, 128), jnp.float32),
        scratch_types=[
            pltpu.VMEM((_RPW, 128), jnp.int32),
            pltpu.VMEM((_RPW, 128), jnp.float32),
            pltpu.SemaphoreType.DMA,
        ],
        compiler_params=pltpu.CompilerParams(use_tc_tiling_on_sc=False),
    )
    def gather_k(tbl_hbm, offs_hbm, out_hbm, offs_v, rows_v, sem):
        wid = lax.axis_index("s") * info.num_cores + lax.axis_index("c")
        base = wid * _RPW
        pltpu.sync_copy(offs_hbm.at[pl.ds(base, _RPW)], offs_v)
        copies = [
            pltpu.async_copy(tbl_hbm.at[offs_v.at[j]], rows_v.at[j], sem)
            for j in range(_RPW)
        ]
        for c in copies:
            c.wait()
        pltpu.sync_copy(rows_v, out_hbm.at[pl.ds(base, _RPW)])

    return gather_k(table_flat, offs)


def _logits_t_block(e_ref, wt_ref, b_ref):
    logits_t = lax.dot_general(
        wt_ref[...].astype(jnp.bfloat16), e_ref[...].astype(jnp.bfloat16),
        (((0,), (0,)), ((), ())),
        preferred_element_type=jnp.float32,
    )
    ones = jnp.ones((_BATCH, 1), dtype=jnp.float32)
    bias_t = lax.dot_general(
        b_ref[...], ones, (((0,), (1,)), ((), ())),
        preferred_element_type=jnp.float32,
    )
    return logits_t + bias_t


def _fused_body(e_ref, wt_ref, b_ref, o_ref, m_s, s_s, lse_s):
    p = pl.program_id(0)
    v = pl.program_id(1)
    logits_t = _logits_t_block(e_ref, wt_ref, b_ref)

    @pl.when(p == 0)
    def _stats_phase():
        @pl.when(v == 0)
        def _():
            m_s[...] = jnp.full_like(m_s, _NEG)
            s_s[...] = jnp.zeros_like(s_s)

        row = v * _VB + lax.broadcasted_iota(jnp.int32, (_VB, 1), 0)
        lm = jnp.where(row < _VOCAB, logits_t, _NEG)
        m_old = m_s[...]
        m_new = jnp.maximum(m_old, jnp.max(lm, axis=0, keepdims=True))
        s_new = s_s[...] * jnp.exp(m_old - m_new) + jnp.sum(
            jnp.exp(lm - m_new), axis=0, keepdims=True)
        m_s[...] = m_new
        s_s[...] = s_new

        @pl.when(v == _NV - 1)
        def _():
            lse_s[...] = m_new + jnp.log(s_new)

    @pl.when(p == 1)
    def _write_phase():
        o_ref[...] = logits_t - lse_s[...]


def kernel(target_word, emb_table, W, b):
    idx = target_word.astype(jnp.int32)
    table_flat = emb_table.T.reshape(_VOCAB * _DIM)
    offs = (jnp.arange(_DIM, dtype=jnp.int32) * _VOCAB)[:, None] + idx[None, :]
    e_t = _sc_gather_t(table_flat, offs.reshape(_NROW, 128))
    e_t = e_t.reshape(_DIM, _BATCH)
    Wt = W.T
    b2 = b.reshape(1, _VOCAB)

    out_t = pl.pallas_call(
        _fused_body,
        grid=(2, _NV),
        in_specs=[
            pl.BlockSpec((_DIM, _BATCH), lambda p, v: (0, 0)),
            pl.BlockSpec((_DIM, _VB), lambda p, v: (0, v)),
            pl.BlockSpec((1, _VB), lambda p, v: (0, v)),
        ],
        out_specs=pl.BlockSpec((_VB, _BATCH), lambda p, v: (p * v, 0)),
        out_shape=jax.ShapeDtypeStruct((_VOCAB, _BATCH), jnp.float32),
        scratch_shapes=[
            pltpu.VMEM((1, _BATCH), jnp.float32),
            pltpu.VMEM((1, _BATCH), jnp.float32),
            pltpu.VMEM((1, _BATCH), jnp.float32),
        ],
        compiler_params=pltpu.CompilerParams(
            dimension_semantics=("arbitrary", "arbitrary")),
    )(e_t, Wt, b2)
    return out_t.T


# speedup vs baseline: 2.8465x; 1.1966x over previous
import functools

import jax
import jax.numpy as jnp
from jax import lax
from jax.experimental import pallas as pl
from jax.experimental.pallas import tpu as pltpu
from jax.experimental.pallas import tpu_sc as plsc

_VOCAB = 100000
_DIM = 64
_BATCH = 1024
_VB = 2048
_NV = (_VOCAB + _VB - 1) // _VB
_NEG = -1e30


_NROW = _DIM * _BATCH // 128
_RPW = _NROW // 32


def _sc_gather_t(table_flat, offs):
    info = plsc.get_sparse_core_info()
    mesh = plsc.VectorSubcoreMesh(core_axis_name="c", subcore_axis_name="s")

    @functools.partial(
        pl.kernel,
        mesh=mesh,
        out_type=jax.ShapeDtypeStruct((_NROW, 128), jnp.float32),
        scratch_types=[
            pltpu.VMEM((_RPW, 128), jnp.int32),
            pltpu.VMEM((_RPW, 128), jnp.float32),
            pltpu.SemaphoreType.DMA,
        ],
        compiler_params=pltpu.CompilerParams(use_tc_tiling_on_sc=False),
    )
    def gather_k(tbl_hbm, offs_hbm, out_hbm, offs_v, rows_v, sem):
        wid = lax.axis_index("s") * info.num_cores + lax.axis_index("c")
        base = wid * _RPW
        pltpu.sync_copy(offs_hbm.at[pl.ds(base, _RPW)], offs_v)
        copies = [
            pltpu.async_copy(tbl_hbm.at[offs_v.at[j]], rows_v.at[j], sem)
            for j in range(_RPW)
        ]
        for c in copies:
            c.wait()
        pltpu.sync_copy(rows_v, out_hbm.at[pl.ds(base, _RPW)])

    return gather_k(table_flat, offs)


_KE = _DIM + 4


def _fused_body(e_ref, wt_ref, b_ref, o_ref, m_s, s_s, wt_s, ee_s):
    p = pl.program_id(0)
    v = pl.program_id(1)

    @pl.when((p == 0) & (v == 0))
    def _prep_e():
        ee_s[pl.ds(0, _DIM), :] = e_ref[...].astype(jnp.bfloat16)
        ee_s[pl.ds(_DIM, 2), :] = jnp.ones((2, _BATCH), jnp.bfloat16)
        ee_s[pl.ds(_DIM + 2, 2), :] = jnp.zeros((2, _BATCH), jnp.bfloat16)

    @pl.when(p == 0)
    def _stats_phase():
        bb = b_ref[...]
        b_hi = bb.astype(jnp.bfloat16)
        b_lo = (bb - b_hi.astype(jnp.float32)).astype(jnp.bfloat16)
        wt_ext = jnp.concatenate(
            [wt_ref[...].astype(jnp.bfloat16), b_hi, b_lo,
             jnp.ones((2, _VB), jnp.bfloat16)], axis=0)
        wt_s[:, pl.ds(v * _VB, _VB)] = wt_ext
        logits = lax.dot_general(
            wt_ext, ee_s[...], (((0,), (0,)), ((), ())),
            preferred_element_type=jnp.float32)

        @pl.when(v == 0)
        def _():
            m_s[...] = jnp.full_like(m_s, _NEG)
            s_s[...] = jnp.zeros_like(s_s)

        def upd(lm):
            m_old = m_s[...]
            m_new = jnp.maximum(m_old, jnp.max(lm, axis=0, keepdims=True))
            eb = jnp.exp(lm - m_new)
            ssum = lax.dot_general(
                jnp.ones((1, _VB), jnp.float32), eb, (((1,), (0,)), ((), ())),
                preferred_element_type=jnp.float32)
            s_s[...] = s_s[...] * jnp.exp(m_old - m_new) + ssum
            m_s[...] = m_new
            return m_new

        @pl.when(v < _NV - 1)
        def _():
            upd(logits)

        @pl.when(v == _NV - 1)
        def _():
            row = v * _VB + lax.broadcasted_iota(jnp.int32, (_VB, 1), 0)
            m_new = upd(jnp.where(row < _VOCAB, logits, _NEG))
            lse = m_new + jnp.log(s_s[...])
            lse_hi = lse.astype(jnp.bfloat16)
            lse_lo = (lse - lse_hi.astype(jnp.float32)).astype(jnp.bfloat16)
            ee_s[pl.ds(_DIM + 2, 1), :] = -lse_hi
            ee_s[pl.ds(_DIM + 3, 1), :] = -lse_lo

    @pl.when(p == 1)
    def _write_phase():
        wt_ext = wt_s[:, pl.ds(v * _VB, _VB)]
        o_ref[...] = lax.dot_general(
            wt_ext, ee_s[...], (((0,), (0,)), ((), ())),
            preferred_element_type=jnp.float32)


def kernel(target_word, emb_table, W, b):
    idx = target_word.astype(jnp.int32)
    table_flat = emb_table.T.reshape(_VOCAB * _DIM)
    offs = (jnp.arange(_DIM, dtype=jnp.int32) * _VOCAB)[:, None] + idx[None, :]
    e_t = _sc_gather_t(table_flat, offs.reshape(_NROW, 128))
    e_t = e_t.reshape(_DIM, _BATCH)
    Wt = W.T
    b2 = b.reshape(1, _VOCAB)

    out_t = pl.pallas_call(
        _fused_body,
        grid=(2, _NV),
        in_specs=[
            pl.BlockSpec((_DIM, _BATCH), lambda p, v: (0, 0)),
            pl.BlockSpec((_DIM, _VB),
                         lambda p, v: (0, jnp.where(p == 0, v, _NV - 1))),
            pl.BlockSpec((1, _VB),
                         lambda p, v: (0, jnp.where(p == 0, v, _NV - 1))),
        ],
        out_specs=pl.BlockSpec((_VB, _BATCH), lambda p, v: (p * v, 0)),
        out_shape=jax.ShapeDtypeStruct((_VOCAB, _BATCH), jnp.float32),
        scratch_shapes=[
            pltpu.VMEM((1, _BATCH), jnp.float32),
            pltpu.VMEM((1, _BATCH), jnp.float32),
            pltpu.VMEM((_KE, _NV * _VB), jnp.bfloat16),
            pltpu.VMEM((_KE, _BATCH), jnp.bfloat16),
        ],
        compiler_params=pltpu.CompilerParams(
            dimension_semantics=("arbitrary", "arbitrary")),
    )(e_t, Wt, b2)
    return out_t.T
